# baseline (speedup 1.0000x reference)
import functools

import jax
import jax.numpy as jnp
from jax.experimental import pallas as pl
from jax.experimental.pallas import tpu as pltpu

N_E = 8192
E_DIM = 32
BETA = 0.25
TM = 256


def _vq_tile_kernel(z_ref, a_ref, b_ref, wneg_ref,
                    d_ref, oh_ref, idx_ref, zq_ref, cnt_ref, ls_ref):
    i = pl.program_id(0)
    z = z_ref[...]
    wneg = wneg_ref[...]
    c2 = jax.lax.dot_general(z, wneg, (((1,), (1,)), ((), ())),
                             preferred_element_type=jnp.float32)
    d = (a_ref[...] + b_ref[...]) + c2
    d_ref[...] = d
    minv = jnp.min(d, axis=1, keepdims=True)
    iota_f = jax.lax.broadcasted_iota(jnp.int32, d.shape, 1).astype(jnp.float32)
    idx_f = jnp.min(jnp.where(d == minv, iota_f, jnp.float32(N_E)),
                    axis=1, keepdims=True)
    idx_ref[...] = idx_f.astype(jnp.int32)
    oh = (iota_f == idx_f).astype(jnp.float32)
    oh_ref[...] = oh
    zq = -0.5 * jax.lax.dot_general(oh, wneg, (((1,), (0,)), ((), ())),
                                    preferred_element_type=jnp.float32,
                                    precision=jax.lax.Precision.HIGHEST)
    zq_ref[...] = zq
    diff = zq - z
    ones_row = jnp.ones((1, TM), dtype=jnp.float32)
    tile_cnt = jax.lax.dot_general(ones_row, oh, (((1,), (0,)), ((), ())),
                                   preferred_element_type=jnp.float32,
                                   precision=jax.lax.Precision.HIGHEST)

    @pl.when(i == 0)
    def _init():
        cnt_ref[...] = jnp.zeros_like(cnt_ref)
        ls_ref[...] = jnp.zeros_like(ls_ref)

    cnt_ref[...] += tile_cnt
    ls_ref[...] += jnp.sum(diff * diff, keepdims=True)


@jax.jit
def kernel(z, emb_w):
    B, C, H, W = z.shape
    M = B * H * W
    z_perm = jnp.transpose(z, (0, 2, 3, 1))
    z_flat = z_perm.reshape(-1, E_DIM)
    a = jnp.sum(z_flat ** 2, axis=1, keepdims=True)
    b = jnp.sum(emb_w ** 2, axis=1)[None, :]
    wneg = -2.0 * emb_w

    grid = (M // TM,)
    d, oh, idx, zq_flat, cnt, ls = pl.pallas_call(
        _vq_tile_kernel,
        grid=grid,
        in_specs=[
            pl.BlockSpec((TM, E_DIM), lambda i: (i, 0)),
            pl.BlockSpec((TM, 1), lambda i: (i, 0)),
            pl.BlockSpec((1, N_E), lambda i: (0, 0)),
            pl.BlockSpec((N_E, E_DIM), lambda i: (0, 0)),
        ],
        out_specs=[
            pl.BlockSpec((TM, N_E), lambda i: (i, 0)),
            pl.BlockSpec((TM, N_E), lambda i: (i, 0)),
            pl.BlockSpec((TM, 1), lambda i: (i, 0)),
            pl.BlockSpec((TM, E_DIM), lambda i: (i, 0)),
            pl.BlockSpec((1, N_E), lambda i: (0, 0)),
            pl.BlockSpec((1, 1), lambda i: (0, 0)),
        ],
        out_shape=[
            jax.ShapeDtypeStruct((M, N_E), jnp.float32),
            jax.ShapeDtypeStruct((M, N_E), jnp.float32),
            jax.ShapeDtypeStruct((M, 1), jnp.int32),
            jax.ShapeDtypeStruct((M, E_DIM), jnp.float32),
            jax.ShapeDtypeStruct((1, N_E), jnp.float32),
            jax.ShapeDtypeStruct((1, 1), jnp.float32),
        ],
        compiler_params=pltpu.CompilerParams(
            dimension_semantics=("arbitrary",)),
    )(z_flat, a, b, wneg)

    loss = (ls[0, 0] / (M * E_DIM)) * (1.0 + BETA)
    e_mean = cnt[0] / M
    perplexity = jnp.exp(-jnp.sum(e_mean * jnp.log(e_mean + 1e-10)))
    z_q = z_flat + (zq_flat - z_flat)
    z_q_out = jnp.transpose(z_q.reshape(B, H, W, C), (0, 3, 1, 2))
    return (z_q_out, loss, perplexity, oh, idx, d)


# speedup vs baseline: 2.1563x; 2.1563x over previous
import functools

import jax
import jax.numpy as jnp
from jax.experimental import pallas as pl
from jax.experimental.pallas import tpu as pltpu

N_E = 8192
E_DIM = 32
BETA = 0.25
TM = 256


def _vq_tile_kernel(z_ref, a_ref, b_ref, wneg_ref, w1_ref, w2_ref, w3_ref,
                    d_ref, oh_ref, idx_ref, zq_ref, cnt_ref, ls_ref):
    i = pl.program_id(0)
    z = z_ref[...]
    wneg = wneg_ref[...]
    c2 = jax.lax.dot_general(z, wneg, (((1,), (1,)), ((), ())),
                             preferred_element_type=jnp.float32)
    d = (a_ref[...] + b_ref[...]) + c2
    d_ref[...] = d
    minv = jnp.min(d, axis=1, keepdims=True)
    iota_f = jax.lax.broadcasted_iota(jnp.int32, d.shape, 1).astype(jnp.float32)
    idx_f = jnp.min(jnp.where(d == minv, iota_f, jnp.float32(N_E)),
                    axis=1, keepdims=True)
    idx_ref[...] = idx_f.astype(jnp.int32)
    oh = jnp.where(iota_f == idx_f, 1.0, 0.0)
    oh_ref[...] = oh
    oh_b = oh.astype(jnp.bfloat16)
    dn = (((1,), (0,)), ((), ()))
    zqn = (jax.lax.dot_general(oh_b, w1_ref[...], dn,
                               preferred_element_type=jnp.float32)
           + jax.lax.dot_general(oh_b, w2_ref[...], dn,
                                 preferred_element_type=jnp.float32)
           + jax.lax.dot_general(oh_b, w3_ref[...], dn,
                                 preferred_element_type=jnp.float32))
    zq = -0.5 * zqn
    zq_ref[...] = zq
    diff = zq - z
    tile_cnt = jnp.sum(oh, axis=0, keepdims=True)

    @pl.when(i == 0)
    def _init():
        cnt_ref[...] = jnp.zeros_like(cnt_ref)
        ls_ref[...] = jnp.zeros_like(ls_ref)

    cnt_ref[...] += tile_cnt
    ls_ref[...] += jnp.sum(diff * diff, keepdims=True)


@jax.jit
def kernel(z, emb_w):
    B, C, H, W = z.shape
    M = B * H * W
    z_perm = jnp.transpose(z, (0, 2, 3, 1))
    z_flat = z_perm.reshape(-1, E_DIM)
    a = jnp.sum(z_flat ** 2, axis=1, keepdims=True)
    b = jnp.sum(emb_w ** 2, axis=1)[None, :]
    wneg = -2.0 * emb_w
    w1 = wneg.astype(jnp.bfloat16)
    r1 = wneg - w1.astype(jnp.float32)
    w2 = r1.astype(jnp.bfloat16)
    w3 = (r1 - w2.astype(jnp.float32)).astype(jnp.bfloat16)

    grid = (M // TM,)
    d, oh, idx, zq_flat, cnt, ls = pl.pallas_call(
        _vq_tile_kernel,
        grid=grid,
        in_specs=[
            pl.BlockSpec((TM, E_DIM), lambda i: (i, 0)),
            pl.BlockSpec((TM, 1), lambda i: (i, 0)),
            pl.BlockSpec((1, N_E), lambda i: (0, 0)),
            pl.BlockSpec((N_E, E_DIM), lambda i: (0, 0)),
            pl.BlockSpec((N_E, E_DIM), lambda i: (0, 0)),
            pl.BlockSpec((N_E, E_DIM), lambda i: (0, 0)),
            pl.BlockSpec((N_E, E_DIM), lambda i: (0, 0)),
        ],
        out_specs=[
            pl.BlockSpec((TM, N_E), lambda i: (i, 0)),
            pl.BlockSpec((TM, N_E), lambda i: (i, 0)),
            pl.BlockSpec((TM, 1), lambda i: (i, 0)),
            pl.BlockSpec((TM, E_DIM), lambda i: (i, 0)),
            pl.BlockSpec((1, N_E), lambda i: (0, 0)),
            pl.BlockSpec((1, 1), lambda i: (0, 0)),
        ],
        out_shape=[
            jax.ShapeDtypeStruct((M, N_E), jnp.float32),
            jax.ShapeDtypeStruct((M, N_E), jnp.float32),
            jax.ShapeDtypeStruct((M, 1), jnp.int32),
            jax.ShapeDtypeStruct((M, E_DIM), jnp.float32),
            jax.ShapeDtypeStruct((1, N_E), jnp.float32),
            jax.ShapeDtypeStruct((1, 1), jnp.float32),
        ],
        compiler_params=pltpu.CompilerParams(
            dimension_semantics=("arbitrary",)),
    )(z_flat, a, b, wneg, w1, w2, w3)

    loss = (ls[0, 0] / (M * E_DIM)) * (1.0 + BETA)
    e_mean = cnt[0] / M
    perplexity = jnp.exp(-jnp.sum(e_mean * jnp.log(e_mean + 1e-10)))
    z_q = z_flat + (zq_flat - z_flat)
    z_q_out = jnp.transpose(z_q.reshape(B, H, W, C), (0, 3, 1, 2))
    return (z_q_out, loss, perplexity, oh, idx, d)


# speedup vs baseline: 2.5167x; 1.1671x over previous
import functools

import jax
import jax.numpy as jnp
from jax import lax
from jax.experimental import pallas as pl
from jax.experimental.pallas import tpu as pltpu, tpu_sc as plsc

N_E = 8192
E_DIM = 32
BETA = 0.25
TM = 256


def _vq_tile_kernel(z_ref, a_ref, b_ref, wneg_ref,
                    d_ref, oh_ref, idx_ref, cnt_ref):
    i = pl.program_id(0)
    z = z_ref[...]
    wneg = wneg_ref[...]
    c2 = jax.lax.dot_general(z, wneg, (((1,), (1,)), ((), ())),
                             preferred_element_type=jnp.float32)
    d = (a_ref[...] + b_ref[...]) + c2
    d_ref[...] = d
    minv = jnp.min(d, axis=1, keepdims=True)
    iota_f = jax.lax.broadcasted_iota(jnp.int32, d.shape, 1).astype(jnp.float32)
    idx_f = jnp.min(jnp.where(d == minv, iota_f, jnp.float32(N_E)),
                    axis=1, keepdims=True)
    idx_ref[...] = idx_f.astype(jnp.int32)
    oh = jnp.where(iota_f == idx_f, 1.0, 0.0)
    oh_ref[...] = oh

    @pl.when(i == 0)
    def _init():
        cnt_ref[...] = jnp.zeros_like(cnt_ref)

    cnt_ref[...] += jnp.sum(oh, axis=0, keepdims=True)


def _make_sc_gather(B, D):
    info = plsc.get_sparse_core_info()
    nw = info.num_cores * info.num_subcores
    b_per_w = B // nw
    mesh = plsc.VectorSubcoreMesh(core_axis_name="c", subcore_axis_name="s")

    @functools.partial(
        pl.kernel, mesh=mesh,
        out_type=jax.ShapeDtypeStruct((B, D), jnp.float32),
        scratch_types=[
            pltpu.VMEM((b_per_w,), jnp.int32),
            pltpu.VMEM((b_per_w, D), jnp.float32),
            pltpu.SemaphoreType.DMA,
        ],
    )
    def gather_kernel(table_hbm, idx_hbm, out_hbm, idx_v, rows_v, sem):
        wid = lax.axis_index("s") * info.num_cores + lax.axis_index("c")
        base = wid * b_per_w
        pltpu.sync_copy(idx_hbm.at[pl.ds(base, b_per_w)], idx_v)
        pltpu.async_copy(table_hbm.at[idx_v], rows_v, sem).wait()
        pltpu.sync_copy(rows_v, out_hbm.at[pl.ds(base, b_per_w)])

    return gather_kernel


@jax.jit
def kernel(z, emb_w):
    B, C, H, W = z.shape
    M = B * H * W
    z_perm = jnp.transpose(z, (0, 2, 3, 1))
    z_flat = z_perm.reshape(-1, E_DIM)
    a = jnp.sum(z_flat ** 2, axis=1, keepdims=True)
    b = jnp.sum(emb_w ** 2, axis=1)[None, :]
    wneg = -2.0 * emb_w

    grid = (M // TM,)
    d, oh, idx, cnt = pl.pallas_call(
        _vq_tile_kernel,
        grid=grid,
        in_specs=[
            pl.BlockSpec((TM, E_DIM), lambda i: (i, 0)),
            pl.BlockSpec((TM, 1), lambda i: (i, 0)),
            pl.BlockSpec((1, N_E), lambda i: (0, 0)),
            pl.BlockSpec((N_E, E_DIM), lambda i: (0, 0)),
        ],
        out_specs=[
            pl.BlockSpec((TM, N_E), lambda i: (i, 0)),
            pl.BlockSpec((TM, N_E), lambda i: (i, 0)),
            pl.BlockSpec((TM, 1), lambda i: (i, 0)),
            pl.BlockSpec((1, N_E), lambda i: (0, 0)),
        ],
        out_shape=[
            jax.ShapeDtypeStruct((M, N_E), jnp.float32),
            jax.ShapeDtypeStruct((M, N_E), jnp.float32),
            jax.ShapeDtypeStruct((M, 1), jnp.int32),
            jax.ShapeDtypeStruct((1, N_E), jnp.float32),
        ],
        compiler_params=pltpu.CompilerParams(
            dimension_semantics=("arbitrary",)),
    )(z_flat, a, b, wneg)

    emb_pad = jnp.pad(emb_w, ((0, 0), (0, 128 - E_DIM)))
    zq_flat = _make_sc_gather(M, 128)(emb_pad, idx[:, 0])[:, :E_DIM]

    loss = (1.0 + BETA) * jnp.mean((zq_flat - z_flat) ** 2)
    e_mean = cnt[0] / M
    perplexity = jnp.exp(-jnp.sum(e_mean * jnp.log(e_mean + 1e-10)))
    z_q = z_flat + (zq_flat - z_flat)
    z_q_out = jnp.transpose(z_q.reshape(B, H, W, C), (0, 3, 1, 2))
    return (z_q_out, loss, perplexity, oh, idx, d)
